# baseline (speedup 1.0000x reference)
import functools

import jax
import jax.numpy as jnp
from jax import lax
from jax.experimental import pallas as pl
from jax.experimental.pallas import tpu as pltpu
from jax.experimental.pallas import tpu_sc as plsc

D_MODEL = 64
SCALE = 8.0

NUM_CORES = 2
NUM_SUBCORES = 16
NUM_WORKERS = NUM_CORES * NUM_SUBCORES

CB = 256
N_COLS = 200
N_BATCH = 4096
CHUNKS_PER_COL = N_BATCH // CB
N_CHUNKS = N_COLS * CHUNKS_PER_COL
PER_TILE = N_CHUNKS // NUM_WORKERS


@jax.jit
def _embed(idx_flat, table):
    mesh = plsc.VectorSubcoreMesh(core_axis_name="c", subcore_axis_name="s")

    @functools.partial(
        pl.kernel,
        mesh=mesh,
        out_type=jax.ShapeDtypeStruct((N_COLS, D_MODEL, N_BATCH), jnp.float32),
        scratch_types=[
            pltpu.VMEM((CB,), jnp.int32),
            pltpu.VMEM((CB,), jnp.int32),
            pltpu.VMEM((CB, D_MODEL), jnp.float32),
            pltpu.VMEM((CB, D_MODEL), jnp.float32),
            pltpu.VMEM((D_MODEL, CB), jnp.float32),
            pltpu.VMEM((D_MODEL, CB), jnp.float32),
            *[pltpu.SemaphoreType.DMA for _ in range(6)],
        ],
        compiler_params=pltpu.CompilerParams(
            use_tc_tiling_on_sc=False, needs_layout_passes=False
        ),
    )
    def emb(idx_hbm, table_hbm, out_hbm,
            idx0, idx1, rows0, rows1, tr0, tr1,
            is0, is1, gs0, gs1, os0, os1):
        idx = (idx0, idx1)
        rows = (rows0, rows1)
        tr = (tr0, tr1)
        isem = (is0, is1)
        gsem = (gs0, gs1)
        osem = (os0, os1)

        wid = lax.axis_index("s") * NUM_CORES + lax.axis_index("c")
        ibase = wid * PER_TILE * CB

        iota = lax.iota(jnp.int32, 16)

        def start_idx(b, i):
            pltpu.async_copy(
                idx_hbm.at[pl.ds(ibase + i * CB, CB)], idx[b], isem[b]
            )

        def wait_idx(b):
            pltpu.make_async_copy(
                idx_hbm.at[pl.ds(ibase, CB)], idx[b], isem[b]
            ).wait()

        def start_gather(b, i):
            del i
            pltpu.async_copy(table_hbm.at[idx[b]], rows[b], gsem[b])

        def wait_gather(b):
            pltpu.make_async_copy(
                out_hbm.at[0, :, pl.ds(0, CB)], rows[b], gsem[b]
            ).wait()

        def transpose_scale(b):
            rb = rows[b]
            tb = tr[b]

            @plsc.parallel_loop(0, D_MODEL * (CB // 16), unroll=4)
            def _(it):
                j = it >> 4
                g = it & 15
                ridx = g * 16 + iota
                cidx = jnp.zeros((16,), jnp.int32) + j
                v = plsc.load_gather(rb, [ridx, cidx])
                tb[j, pl.ds(g * 16, 16)] = v * SCALE

        def start_store(b, i):
            ig = wid * PER_TILE + i
            c = ig // CHUNKS_PER_COL
            k = ig % CHUNKS_PER_COL
            pltpu.async_copy(
                tr[b], out_hbm.at[c, :, pl.ds(k * CB, CB)], osem[b]
            )

        def wait_store(b):
            pltpu.make_async_copy(
                tr[b], out_hbm.at[0, :, pl.ds(0, CB)], osem[b]
            ).wait()

        pltpu.sync_copy(idx_hbm.at[pl.ds(ibase, CB)], idx0)
        start_idx(1, 1)
        start_gather(0, 0)

        def chunk_body(i, b, first, do_next_gather, do_idx_load):
            wait_gather(b)
            if do_next_gather:
                wait_idx(1 - b)
                start_gather(1 - b, i + 1)
            if do_idx_load:
                start_idx(b, i + 2)
            if not first:
                wait_store(b)
            transpose_scale(b)
            start_store(b, i)

        chunk_body(0, 0, True, True, True)
        chunk_body(1, 1, True, True, True)

        def outer(p, carry):
            i0 = p * 2
            chunk_body(i0, 0, False, True, True)
            chunk_body(i0 + 1, 1, False, True, True)
            return carry

        lax.fori_loop(1, PER_TILE // 2 - 1, outer, 0)

        chunk_body(PER_TILE - 2, 0, False, True, False)
        chunk_body(PER_TILE - 1, 1, False, False, False)

        wait_store(0)
        wait_store(1)

    return emb(idx_flat, table)


def kernel(x, table):
    xT = x.T
    idx_flat = xT.reshape(N_COLS * N_BATCH).astype(jnp.int32)
    outP = _embed(idx_flat, table)
    return outP.transpose(2, 0, 1)


# speedup vs baseline: 1.6725x; 1.6725x over previous
import functools

import jax
import jax.numpy as jnp
from jax import lax
from jax.experimental import pallas as pl
from jax.experimental.pallas import tpu as pltpu
from jax.experimental.pallas import tpu_sc as plsc

D_MODEL = 64
SCALE = 8.0

NUM_CORES = 2
NUM_SUBCORES = 16
NUM_WORKERS = NUM_CORES * NUM_SUBCORES

CB = 256
N_COLS = 200
N_BATCH = 4096
CHUNKS_PER_COL = N_BATCH // CB
N_CHUNKS = N_COLS * CHUNKS_PER_COL
PER_TILE = N_CHUNKS // NUM_WORKERS


@jax.jit
def _embed(idx_flat, table):
    mesh = plsc.VectorSubcoreMesh(core_axis_name="c", subcore_axis_name="s")

    @functools.partial(
        pl.kernel,
        mesh=mesh,
        out_type=jax.ShapeDtypeStruct((N_COLS, D_MODEL, N_BATCH), jnp.float32),
        scratch_types=[
            pltpu.VMEM((CB,), jnp.int32),
            pltpu.VMEM((CB,), jnp.int32),
            pltpu.VMEM((CB, D_MODEL), jnp.float32),
            pltpu.VMEM((CB, D_MODEL), jnp.float32),
            pltpu.VMEM((D_MODEL, CB), jnp.float32),
            pltpu.VMEM((D_MODEL, CB), jnp.float32),
            *[pltpu.SemaphoreType.DMA for _ in range(6)],
        ],
        compiler_params=pltpu.CompilerParams(
            use_tc_tiling_on_sc=False, needs_layout_passes=False
        ),
    )
    def emb(idx_hbm, table_hbm, out_hbm,
            idx0, idx1, rows0, rows1, tr0, tr1,
            is0, is1, gs0, gs1, os0, os1):
        idx = (idx0, idx1)
        rows = (rows0, rows1)
        tr = (tr0, tr1)
        isem = (is0, is1)
        gsem = (gs0, gs1)
        osem = (os0, os1)

        wid = lax.axis_index("s") * NUM_CORES + lax.axis_index("c")
        ibase = wid * PER_TILE * CB

        iota = lax.iota(jnp.int32, 16)

        def start_idx(b, i):
            pltpu.async_copy(
                idx_hbm.at[pl.ds(ibase + i * CB, CB)], idx[b], isem[b]
            )

        def wait_idx(b):
            pltpu.make_async_copy(
                idx_hbm.at[pl.ds(ibase, CB)], idx[b], isem[b]
            ).wait()

        def start_gather(b, i):
            del i
            pltpu.async_copy(table_hbm.at[idx[b]], rows[b], gsem[b])

        def wait_gather(b):
            pltpu.make_async_copy(
                out_hbm.at[0, :, pl.ds(0, CB)], rows[b], gsem[b]
            ).wait()

        def transpose_scale(b):
            rb = rows[b]
            tb = tr[b]

            @plsc.parallel_loop(0, (D_MODEL // 16) * CB, unroll=4)
            def _(it):
                s = it & 15
                gb = (it >> 4) & (CB // 16 - 1)
                jb = it >> 8
                rr = gb * 16 + ((iota + s) & 15)
                cc = jb * 16 + iota
                v = plsc.load_gather(rb, [rr, cc])
                plsc.store_scatter(tb, [cc, rr], v * SCALE)

        def start_store(b, i):
            ig = wid * PER_TILE + i
            c = ig // CHUNKS_PER_COL
            k = ig % CHUNKS_PER_COL
            pltpu.async_copy(
                tr[b], out_hbm.at[c, :, pl.ds(k * CB, CB)], osem[b]
            )

        def wait_store(b):
            pltpu.make_async_copy(
                tr[b], out_hbm.at[0, :, pl.ds(0, CB)], osem[b]
            ).wait()

        pltpu.sync_copy(idx_hbm.at[pl.ds(ibase, CB)], idx0)
        start_idx(1, 1)
        start_gather(0, 0)

        def chunk_body(i, b, first, do_next_gather, do_idx_load):
            wait_gather(b)
            if do_next_gather:
                wait_idx(1 - b)
                start_gather(1 - b, i + 1)
            if do_idx_load:
                start_idx(b, i + 2)
            if not first:
                wait_store(b)
            transpose_scale(b)
            start_store(b, i)

        chunk_body(0, 0, True, True, True)
        chunk_body(1, 1, True, True, True)

        def outer(p, carry):
            i0 = p * 2
            chunk_body(i0, 0, False, True, True)
            chunk_body(i0 + 1, 1, False, True, True)
            return carry

        lax.fori_loop(1, PER_TILE // 2 - 1, outer, 0)

        chunk_body(PER_TILE - 2, 0, False, True, False)
        chunk_body(PER_TILE - 1, 1, False, False, False)

        wait_store(0)
        wait_store(1)

    return emb(idx_flat, table)


def kernel(x, table):
    xT = x.T
    idx_flat = xT.reshape(N_COLS * N_BATCH).astype(jnp.int32)
    outP = _embed(idx_flat, table)
    return outP.transpose(2, 0, 1)


# speedup vs baseline: 2.5187x; 1.5060x over previous
import functools

import jax
import jax.numpy as jnp
from jax import lax
from jax.experimental import pallas as pl
from jax.experimental.pallas import tpu as pltpu
from jax.experimental.pallas import tpu_sc as plsc

D_MODEL = 64
SCALE = 8.0

NUM_CORES = 2
NUM_SUBCORES = 16
NUM_WORKERS = NUM_CORES * NUM_SUBCORES

CB = 256
N_COLS = 200
N_BATCH = 4096
CHUNKS_PER_COL = N_BATCH // CB
N_CHUNKS = N_COLS * CHUNKS_PER_COL
PER_TILE = N_CHUNKS // NUM_WORKERS


VOCAB = 1000000
TC_BW = 8192
TC_GRID = -(-VOCAB // TC_BW)
T2_ROWS = (TC_GRID - 1) * (TC_BW // 2) + (VOCAB - (TC_GRID - 1) * TC_BW)


def _tc_transpose_scale(tableT):
    def body(x_ref, o_ref):
        t = x_ref[...].T * SCALE
        lo = t[: TC_BW // 2, :]
        hi = t[TC_BW // 2 :, :]
        o_ref[...] = jnp.concatenate([lo, hi], axis=1)

    return pl.pallas_call(
        body,
        grid=(TC_GRID,),
        in_specs=[pl.BlockSpec((D_MODEL, TC_BW), lambda i: (0, i))],
        out_specs=pl.BlockSpec((TC_BW // 2, 128), lambda i: (i, 0)),
        out_shape=jax.ShapeDtypeStruct((T2_ROWS, 128), jnp.float32),
    )(tableT)


@jax.jit
def _embed(idx_flat, table):
    mesh = plsc.VectorSubcoreMesh(core_axis_name="c", subcore_axis_name="s")

    @functools.partial(
        pl.kernel,
        mesh=mesh,
        out_type=jax.ShapeDtypeStruct((N_COLS, D_MODEL, N_BATCH), jnp.float32),
        scratch_types=[
            pltpu.VMEM((CB,), jnp.int32),
            pltpu.VMEM((CB,), jnp.int32),
            pltpu.VMEM((CB, D_MODEL), jnp.float32),
            pltpu.VMEM((CB, D_MODEL), jnp.float32),
            pltpu.VMEM((D_MODEL, CB), jnp.float32),
            pltpu.VMEM((D_MODEL, CB), jnp.float32),
            *[pltpu.SemaphoreType.DMA for _ in range(6)],
        ],
        compiler_params=pltpu.CompilerParams(
            use_tc_tiling_on_sc=False, needs_layout_passes=False
        ),
    )
    def emb(idx_hbm, table_hbm, out_hbm,
            idx0, idx1, rows0, rows1, tr0, tr1,
            is0, is1, gs0, gs1, os0, os1):
        idx = (idx0, idx1)
        rows = (rows0, rows1)
        tr = (tr0, tr1)
        isem = (is0, is1)
        gsem = (gs0, gs1)
        osem = (os0, os1)

        wid = lax.axis_index("s") * NUM_CORES + lax.axis_index("c")
        ibase = wid * PER_TILE * CB

        iota = lax.iota(jnp.int32, 16)

        def start_idx(b, i):
            pltpu.async_copy(
                idx_hbm.at[pl.ds(ibase + i * CB, CB)], idx[b], isem[b]
            )

        def wait_idx(b):
            pltpu.make_async_copy(
                idx_hbm.at[pl.ds(ibase, CB)], idx[b], isem[b]
            ).wait()

        def start_gather(b, i):
            del i
            pltpu.async_copy(table_hbm.at[idx[b]], rows[b], gsem[b])

        def wait_gather(b):
            pltpu.make_async_copy(
                out_hbm.at[0, :, pl.ds(0, CB)], rows[b], gsem[b]
            ).wait()

        def transpose_scale(b):
            rb = rows[b]
            tb = tr[b]

            @plsc.parallel_loop(0, (D_MODEL // 16) * CB, unroll=4)
            def _(it):
                s = it & 15
                gb = (it >> 4) & (CB // 16 - 1)
                jb = it >> 8
                rr = gb * 16 + ((iota + s) & 15)
                cc = jb * 16 + iota
                v = plsc.load_gather(rb, [rr, cc])
                plsc.store_scatter(tb, [cc, rr], v)

        def start_store(b, i):
            ig = wid * PER_TILE + i
            c = ig // CHUNKS_PER_COL
            k = ig % CHUNKS_PER_COL
            pltpu.async_copy(
                tr[b], out_hbm.at[c, :, pl.ds(k * CB, CB)], osem[b]
            )

        def wait_store(b):
            pltpu.make_async_copy(
                tr[b], out_hbm.at[0, :, pl.ds(0, CB)], osem[b]
            ).wait()

        pltpu.sync_copy(idx_hbm.at[pl.ds(ibase, CB)], idx0)
        start_idx(1, 1)
        start_gather(0, 0)

        def chunk_body(i, b, first, do_next_gather, do_idx_load):
            wait_gather(b)
            if do_next_gather:
                wait_idx(1 - b)
                start_gather(1 - b, i + 1)
            if do_idx_load:
                start_idx(b, i + 2)
            if not first:
                wait_store(b)
            transpose_scale(b)
            start_store(b, i)

        chunk_body(0, 0, True, True, True)
        chunk_body(1, 1, True, True, True)

        def outer(p, carry):
            i0 = p * 2
            chunk_body(i0, 0, False, True, True)
            chunk_body(i0 + 1, 1, False, True, True)
            return carry

        lax.fori_loop(1, PER_TILE // 2 - 1, outer, 0)

        chunk_body(PER_TILE - 2, 0, False, True, False)
        chunk_body(PER_TILE - 1, 1, False, False, False)

        wait_store(0)
        wait_store(1)

    return emb(idx_flat, table)


@jax.jit
def _full(x, table):
    xT = x.T
    v = xT.reshape(N_COLS * N_BATCH).astype(jnp.int32)
    idx_flat = (v & -TC_BW) + ((v & (TC_BW // 2 - 1)) << 1) + ((v >> 12) & 1)
    table_rm = _tc_transpose_scale(table.T).reshape(2 * T2_ROWS, D_MODEL)
    outP = _embed(idx_flat, table_rm)
    return outP.transpose(2, 0, 1)


def kernel(x, table):
    return _full(x, table)


# speedup vs baseline: 3.7784x; 1.5001x over previous
import functools

import jax
import jax.numpy as jnp
from jax import lax
from jax.experimental import pallas as pl
from jax.experimental.pallas import tpu as pltpu
from jax.experimental.pallas import tpu_sc as plsc

D_MODEL = 64
SCALE = 8.0

NUM_CORES = 2
NUM_SUBCORES = 16
NUM_WORKERS = NUM_CORES * NUM_SUBCORES

CB = 256
N_COLS = 200
N_BATCH = 4096
CHUNKS_PER_COL = N_BATCH // CB
N_CHUNKS = N_COLS * CHUNKS_PER_COL
PER_TILE = N_CHUNKS // NUM_WORKERS


VOCAB = 1000000
TC_BW = 8192
TC_GRID = -(-VOCAB // TC_BW)
T2_ROWS = (TC_GRID - 1) * (TC_BW // 2) + (VOCAB - (TC_GRID - 1) * TC_BW)


def _tc_transpose_scale(tableT):
    def body(x_ref, o_ref):
        t = x_ref[...].T * SCALE
        lo = t[: TC_BW // 2, :]
        hi = t[TC_BW // 2 :, :]
        o_ref[...] = jnp.concatenate([lo, hi], axis=1)

    return pl.pallas_call(
        body,
        grid=(TC_GRID,),
        in_specs=[pl.BlockSpec((D_MODEL, TC_BW), lambda i: (0, i))],
        out_specs=pl.BlockSpec((TC_BW // 2, 128), lambda i: (i, 0)),
        out_shape=jax.ShapeDtypeStruct((T2_ROWS, 128), jnp.float32),
    )(tableT)


GPC = CB // 128
N_G = N_BATCH // 128


@jax.jit
def _embed(idx_flat, table):
    mesh = plsc.VectorSubcoreMesh(core_axis_name="c", subcore_axis_name="s")

    @functools.partial(
        pl.kernel,
        mesh=mesh,
        out_type=jax.ShapeDtypeStruct((N_COLS, 8, N_G, 1024), jnp.float32),
        scratch_types=[
            pltpu.VMEM((CB,), jnp.int32),
            pltpu.VMEM((CB,), jnp.int32),
            pltpu.VMEM((CB, D_MODEL), jnp.float32),
            pltpu.VMEM((CB, D_MODEL), jnp.float32),
            pltpu.VMEM((8 * GPC, 1024), jnp.float32),
            pltpu.VMEM((8 * GPC, 1024), jnp.float32),
            *[pltpu.SemaphoreType.DMA for _ in range(6)],
        ],
        compiler_params=pltpu.CompilerParams(
            use_tc_tiling_on_sc=False, needs_layout_passes=False
        ),
    )
    def emb(idx_hbm, table_hbm, out_hbm,
            idx0, idx1, rows0, rows1, tr0, tr1,
            is0, is1, gs0, gs1, os0, os1):
        idx = (idx0, idx1)
        rows = (rows0, rows1)
        tr = (tr0, tr1)
        isem = (is0, is1)
        gsem = (gs0, gs1)
        osem = (os0, os1)

        wid = lax.axis_index("s") * NUM_CORES + lax.axis_index("c")
        ibase = wid * PER_TILE * CB

        iota = lax.iota(jnp.int32, 16)
        colbase = (iota & 7) << 7
        iotahi = iota >> 3

        def start_idx(b, i):
            pltpu.async_copy(
                idx_hbm.at[pl.ds(ibase + i * CB, CB)], idx[b], isem[b]
            )

        def wait_idx(b):
            pltpu.make_async_copy(
                idx_hbm.at[pl.ds(ibase, CB)], idx[b], isem[b]
            ).wait()

        def start_gather(b, i):
            del i
            pltpu.async_copy(table_hbm.at[idx[b]], rows[b], gsem[b])

        def wait_gather(b):
            pltpu.make_async_copy(
                out_hbm.at[0, 0, pl.ds(0, 16), :], rows[b], gsem[b]
            ).wait()

        def transpose_scale(b):
            rb = rows[b]
            tb = tr[b]

            @plsc.parallel_loop(0, (D_MODEL // 16) * CB, unroll=4)
            def _(it):
                s = it & 15
                gb = (it >> 4) & (CB // 16 - 1)
                jb = it >> 8
                skew = (iota + s) & 15
                rr = gb * 16 + skew
                cc = jb * 16 + iota
                v = plsc.load_gather(rb, [rr, cc])
                row = (gb >> 3) * 8 + jb * 2 + iotahi
                col = colbase + ((gb & 7) * 16 + skew)
                plsc.store_scatter(tb, [row, col], v)

        def start_store(b, i):
            ig = wid * PER_TILE + i
            c = ig // CHUNKS_PER_COL
            k = ig % CHUNKS_PER_COL
            for g in range(GPC):
                pltpu.async_copy(
                    tr[b].at[pl.ds(g * 8, 8), :],
                    out_hbm.at[c, :, GPC * k + g, :],
                    osem[b],
                )

        def wait_store(b):
            for _ in range(GPC):
                pltpu.make_async_copy(
                    tr[b].at[pl.ds(0, 8), :], out_hbm.at[0, :, 0, :], osem[b]
                ).wait()

        pltpu.sync_copy(idx_hbm.at[pl.ds(ibase, CB)], idx0)
        start_idx(1, 1)
        start_gather(0, 0)

        def chunk_body(i, b, first, do_next_gather, do_idx_load):
            wait_gather(b)
            if do_next_gather:
                wait_idx(1 - b)
                start_gather(1 - b, i + 1)
            if do_idx_load:
                start_idx(b, i + 2)
            if not first:
                wait_store(b)
            transpose_scale(b)
            start_store(b, i)

        chunk_body(0, 0, True, True, True)
        chunk_body(1, 1, True, True, True)

        def outer(p, carry):
            i0 = p * 2
            chunk_body(i0, 0, False, True, True)
            chunk_body(i0 + 1, 1, False, True, True)
            return carry

        lax.fori_loop(1, PER_TILE // 2 - 1, outer, 0)

        chunk_body(PER_TILE - 2, 0, False, True, False)
        chunk_body(PER_TILE - 1, 1, False, False, False)

        wait_store(0)
        wait_store(1)

    return emb(idx_flat, table)


@jax.jit
def _full(x, table):
    xT = x.T
    v = xT.reshape(N_COLS * N_BATCH).astype(jnp.int32)
    idx_flat = (v & -TC_BW) + ((v & (TC_BW // 2 - 1)) << 1) + ((v >> 12) & 1)
    table_rm = _tc_transpose_scale(table.T).reshape(2 * T2_ROWS, D_MODEL)
    outP = _embed(idx_flat, table_rm)
    y = (
        outP.reshape(N_COLS, 8, N_G, 8, 128)
        .transpose(0, 1, 3, 2, 4)
        .reshape(N_COLS, D_MODEL, N_BATCH)
    )
    return y.transpose(2, 0, 1)


def kernel(x, table):
    return _full(x, table)


# speedup vs baseline: 4.0532x; 1.0727x over previous
import functools

import jax
import jax.numpy as jnp
from jax import lax
from jax.experimental import pallas as pl
from jax.experimental.pallas import tpu as pltpu
from jax.experimental.pallas import tpu_sc as plsc

D_MODEL = 64
SCALE = 8.0

NUM_CORES = 2
NUM_SUBCORES = 16
NUM_WORKERS = NUM_CORES * NUM_SUBCORES

CB = 256
N_COLS = 200
N_BATCH = 4096
CHUNKS_PER_COL = N_BATCH // CB
N_CHUNKS = N_COLS * CHUNKS_PER_COL
PER_TILE = N_CHUNKS // NUM_WORKERS


VOCAB = 1000000
TC_BW = 16384
TC_GRID = -(-VOCAB // TC_BW)
T2_ROWS = (TC_GRID - 1) * (TC_BW // 2) + (VOCAB - (TC_GRID - 1) * TC_BW)


def _tc_transpose_scale(tableT):
    def body(x_ref, o_ref):
        t = x_ref[...].T * SCALE
        lo = t[: TC_BW // 2, :]
        hi = t[TC_BW // 2 :, :]
        o_ref[...] = jnp.concatenate([lo, hi], axis=1)

    return pl.pallas_call(
        body,
        grid=(TC_GRID,),
        in_specs=[pl.BlockSpec((D_MODEL, TC_BW), lambda i: (0, i))],
        out_specs=pl.BlockSpec((TC_BW // 2, 128), lambda i: (i, 0)),
        out_shape=jax.ShapeDtypeStruct((T2_ROWS, 128), jnp.float32),
    )(tableT)


GPC = CB // 128
N_G = N_BATCH // 128


@jax.jit
def _embed(idx_flat, table):
    mesh = plsc.VectorSubcoreMesh(core_axis_name="c", subcore_axis_name="s")

    @functools.partial(
        pl.kernel,
        mesh=mesh,
        out_type=jax.ShapeDtypeStruct((N_COLS, 8, N_G, 1024), jnp.float32),
        scratch_types=[
            pltpu.VMEM((CB,), jnp.int32),
            pltpu.VMEM((CB,), jnp.int32),
            pltpu.VMEM((CB, D_MODEL), jnp.float32),
            pltpu.VMEM((CB, D_MODEL), jnp.float32),
            pltpu.VMEM((8 * GPC, 1024), jnp.float32),
            pltpu.VMEM((8 * GPC, 1024), jnp.float32),
            *[pltpu.SemaphoreType.DMA for _ in range(6)],
        ],
        compiler_params=pltpu.CompilerParams(
            use_tc_tiling_on_sc=False, needs_layout_passes=False
        ),
    )
    def emb(idx_hbm, table_hbm, out_hbm,
            idx0, idx1, rows0, rows1, tr0, tr1,
            is0, is1, gs0, gs1, os0, os1):
        idx = (idx0, idx1)
        rows = (rows0, rows1)
        tr = (tr0, tr1)
        isem = (is0, is1)
        gsem = (gs0, gs1)
        osem = (os0, os1)

        wid = lax.axis_index("s") * NUM_CORES + lax.axis_index("c")
        ibase = wid * PER_TILE * CB

        iota = lax.iota(jnp.int32, 16)
        colbase = (iota & 7) << 7
        iotahi = iota >> 3

        def start_idx(b, i):
            pltpu.async_copy(
                idx_hbm.at[pl.ds(ibase + i * CB, CB)], idx[b], isem[b]
            )

        def wait_idx(b):
            pltpu.make_async_copy(
                idx_hbm.at[pl.ds(ibase, CB)], idx[b], isem[b]
            ).wait()

        def start_gather(b, i):
            del i
            pltpu.async_copy(table_hbm.at[idx[b]], rows[b], gsem[b])

        def wait_gather(b):
            pltpu.make_async_copy(
                out_hbm.at[0, 0, pl.ds(0, 16), :], rows[b], gsem[b]
            ).wait()

        def transpose_scale(b):
            rb = rows[b]
            tb = tr[b]

            @plsc.parallel_loop(0, (D_MODEL // 16) * CB, unroll=4)
            def _(it):
                s = it & 15
                gb = (it >> 4) & (CB // 16 - 1)
                jb = it >> 8
                skew = (iota + s) & 15
                rr = gb * 16 + skew
                cc = jb * 16 + iota
                v = plsc.load_gather(rb, [rr, cc])
                row = (gb >> 3) * 8 + jb * 2 + iotahi
                col = colbase + ((gb & 7) * 16 + skew)
                plsc.store_scatter(tb, [row, col], v)

        def start_store(b, i):
            ig = wid * PER_TILE + i
            c = ig // CHUNKS_PER_COL
            k = ig % CHUNKS_PER_COL
            for g in range(GPC):
                pltpu.async_copy(
                    tr[b].at[pl.ds(g * 8, 8), :],
                    out_hbm.at[c, :, GPC * k + g, :],
                    osem[b],
                )

        def wait_store(b):
            for _ in range(GPC):
                pltpu.make_async_copy(
                    tr[b].at[pl.ds(0, 8), :], out_hbm.at[0, :, 0, :], osem[b]
                ).wait()

        pltpu.sync_copy(idx_hbm.at[pl.ds(ibase, CB)], idx0)
        start_idx(1, 1)
        start_gather(0, 0)

        def chunk_body(i, b, first, do_next_gather, do_idx_load):
            wait_gather(b)
            if do_next_gather:
                wait_idx(1 - b)
                start_gather(1 - b, i + 1)
            if do_idx_load:
                start_idx(b, i + 2)
            if not first:
                wait_store(b)
            transpose_scale(b)
            start_store(b, i)

        chunk_body(0, 0, True, True, True)
        chunk_body(1, 1, True, True, True)

        def outer(p, carry):
            i0 = p * 2
            chunk_body(i0, 0, False, True, True)
            chunk_body(i0 + 1, 1, False, True, True)
            return carry

        lax.fori_loop(1, PER_TILE // 2 - 1, outer, 0)

        chunk_body(PER_TILE - 2, 0, False, True, False)
        chunk_body(PER_TILE - 1, 1, False, False, False)

        wait_store(0)
        wait_store(1)

    return emb(idx_flat, table)


@jax.jit
def _full(x, table):
    xT = x.T
    v = xT.reshape(N_COLS * N_BATCH).astype(jnp.int32)
    half_sh = (TC_BW // 2).bit_length() - 1
    idx_flat = (
        (v & -TC_BW) + ((v & (TC_BW // 2 - 1)) << 1) + ((v >> half_sh) & 1)
    )
    table_rm = _tc_transpose_scale(table.T).reshape(2 * T2_ROWS, D_MODEL)
    outP = _embed(idx_flat, table_rm)
    y = (
        outP.reshape(N_COLS, 8, N_G, 8, 128)
        .transpose(0, 1, 3, 2, 4)
        .reshape(N_COLS, D_MODEL, N_BATCH)
    )
    return y.transpose(2, 0, 1)


def kernel(x, table):
    return _full(x, table)


# speedup vs baseline: 4.1797x; 1.0312x over previous
import functools

import jax
import jax.numpy as jnp
from jax import lax
from jax.experimental import pallas as pl
from jax.experimental.pallas import tpu as pltpu
from jax.experimental.pallas import tpu_sc as plsc

D_MODEL = 64
SCALE = 8.0

NUM_CORES = 2
NUM_SUBCORES = 16
NUM_WORKERS = NUM_CORES * NUM_SUBCORES

CB = 256
N_COLS = 200
N_BATCH = 4096
CHUNKS_PER_COL = N_BATCH // CB
N_CHUNKS = N_COLS * CHUNKS_PER_COL
PER_TILE = N_CHUNKS // NUM_WORKERS


VOCAB = 1000000
TC_BW = 32768
TC_GRID = -(-VOCAB // TC_BW)
T2_ROWS = (TC_GRID - 1) * (TC_BW // 2) + (VOCAB - (TC_GRID - 1) * TC_BW)


def _tc_transpose_scale(tableT):
    def body(x_ref, o_ref):
        t = x_ref[...].T * SCALE
        lo = t[: TC_BW // 2, :]
        hi = t[TC_BW // 2 :, :]
        o_ref[...] = jnp.concatenate([lo, hi], axis=1)

    return pl.pallas_call(
        body,
        grid=(TC_GRID,),
        in_specs=[pl.BlockSpec((D_MODEL, TC_BW), lambda i: (0, i))],
        out_specs=pl.BlockSpec((TC_BW // 2, 128), lambda i: (i, 0)),
        out_shape=jax.ShapeDtypeStruct((T2_ROWS, 128), jnp.float32),
    )(tableT)


GPC = CB // 128
N_G = N_BATCH // 128


@jax.jit
def _embed(idx_flat, table):
    mesh = plsc.VectorSubcoreMesh(core_axis_name="c", subcore_axis_name="s")

    @functools.partial(
        pl.kernel,
        mesh=mesh,
        out_type=jax.ShapeDtypeStruct((N_COLS, 8, N_G, 1024), jnp.float32),
        scratch_types=[
            pltpu.VMEM((CB,), jnp.int32),
            pltpu.VMEM((CB,), jnp.int32),
            pltpu.VMEM((CB, D_MODEL), jnp.float32),
            pltpu.VMEM((CB, D_MODEL), jnp.float32),
            pltpu.VMEM((8 * GPC, 1024), jnp.float32),
            pltpu.VMEM((8 * GPC, 1024), jnp.float32),
            *[pltpu.SemaphoreType.DMA for _ in range(6)],
        ],
        compiler_params=pltpu.CompilerParams(
            use_tc_tiling_on_sc=False, needs_layout_passes=False
        ),
    )
    def emb(idx_hbm, table_hbm, out_hbm,
            idx0, idx1, rows0, rows1, tr0, tr1,
            is0, is1, gs0, gs1, os0, os1):
        idx = (idx0, idx1)
        rows = (rows0, rows1)
        tr = (tr0, tr1)
        isem = (is0, is1)
        gsem = (gs0, gs1)
        osem = (os0, os1)

        wid = lax.axis_index("s") * NUM_CORES + lax.axis_index("c")
        ibase = wid * PER_TILE * CB

        iota = lax.iota(jnp.int32, 16)
        colbase = (iota & 7) << 7
        iotahi = iota >> 3

        def start_idx(b, i):
            pltpu.async_copy(
                idx_hbm.at[pl.ds(ibase + i * CB, CB)], idx[b], isem[b]
            )

        def wait_idx(b):
            pltpu.make_async_copy(
                idx_hbm.at[pl.ds(ibase, CB)], idx[b], isem[b]
            ).wait()

        def start_gather(b, i):
            del i
            pltpu.async_copy(table_hbm.at[idx[b]], rows[b], gsem[b])

        def wait_gather(b):
            pltpu.make_async_copy(
                out_hbm.at[0, 0, pl.ds(0, 16), :], rows[b], gsem[b]
            ).wait()

        def transpose_scale(b):
            rb = rows[b]
            tb = tr[b]

            @plsc.parallel_loop(0, (D_MODEL // 16) * CB, unroll=4)
            def _(it):
                s = it & 15
                gb = (it >> 4) & (CB // 16 - 1)
                jb = it >> 8
                skew = (iota + s) & 15
                rr = gb * 16 + skew
                cc = jb * 16 + iota
                v = plsc.load_gather(rb, [rr, cc])
                row = (gb >> 3) * 8 + jb * 2 + iotahi
                col = colbase + ((gb & 7) * 16 + skew)
                plsc.store_scatter(tb, [row, col], v)

        def start_store(b, i):
            ig = wid * PER_TILE + i
            c = ig // CHUNKS_PER_COL
            k = ig % CHUNKS_PER_COL
            for g in range(GPC):
                pltpu.async_copy(
                    tr[b].at[pl.ds(g * 8, 8), :],
                    out_hbm.at[c, :, GPC * k + g, :],
                    osem[b],
                )

        def wait_store(b):
            for _ in range(GPC):
                pltpu.make_async_copy(
                    tr[b].at[pl.ds(0, 8), :], out_hbm.at[0, :, 0, :], osem[b]
                ).wait()

        pltpu.sync_copy(idx_hbm.at[pl.ds(ibase, CB)], idx0)
        start_idx(1, 1)
        start_gather(0, 0)

        def chunk_body(i, b, first, do_next_gather, do_idx_load):
            wait_gather(b)
            if do_next_gather:
                wait_idx(1 - b)
                start_gather(1 - b, i + 1)
            if do_idx_load:
                start_idx(b, i + 2)
            if not first:
                wait_store(b)
            transpose_scale(b)
            start_store(b, i)

        chunk_body(0, 0, True, True, True)
        chunk_body(1, 1, True, True, True)

        def outer(p, carry):
            i0 = p * 2
            chunk_body(i0, 0, False, True, True)
            chunk_body(i0 + 1, 1, False, True, True)
            return carry

        lax.fori_loop(1, PER_TILE // 2 - 1, outer, 0)

        chunk_body(PER_TILE - 2, 0, False, True, False)
        chunk_body(PER_TILE - 1, 1, False, False, False)

        wait_store(0)
        wait_store(1)

    return emb(idx_flat, table)


@jax.jit
def _full(x, table):
    xT = x.T
    v = xT.reshape(N_COLS * N_BATCH).astype(jnp.int32)
    half_sh = (TC_BW // 2).bit_length() - 1
    idx_flat = (
        (v & -TC_BW) + ((v & (TC_BW // 2 - 1)) << 1) + ((v >> half_sh) & 1)
    )
    table_rm = _tc_transpose_scale(table.T).reshape(2 * T2_ROWS, D_MODEL)
    outP = _embed(idx_flat, table_rm)
    y = (
        outP.reshape(N_COLS, 8, N_G, 8, 128)
        .transpose(0, 1, 3, 2, 4)
        .reshape(N_COLS, D_MODEL, N_BATCH)
    )
    return y.transpose(2, 0, 1)


def kernel(x, table):
    return _full(x, table)


# speedup vs baseline: 4.1801x; 1.0001x over previous
import functools

import jax
import jax.numpy as jnp
from jax import lax
from jax.experimental import pallas as pl
from jax.experimental.pallas import tpu as pltpu
from jax.experimental.pallas import tpu_sc as plsc

D_MODEL = 64
SCALE = 8.0

NUM_CORES = 2
NUM_SUBCORES = 16
NUM_WORKERS = NUM_CORES * NUM_SUBCORES

CB = 256
N_COLS = 200
N_BATCH = 4096
CHUNKS_PER_COL = N_BATCH // CB
N_CHUNKS = N_COLS * CHUNKS_PER_COL
PER_TILE = N_CHUNKS // NUM_WORKERS


VOCAB = 1000000
TC_BW = 32768
TC_GRID = -(-VOCAB // TC_BW)
T2_ROWS = (TC_GRID - 1) * (TC_BW // 2) + (VOCAB - (TC_GRID - 1) * TC_BW)


def _tc_transpose_scale(tableT):
    def body(x_ref, o_ref):
        t = x_ref[...].T * SCALE
        lo = t[: TC_BW // 2, :]
        hi = t[TC_BW // 2 :, :]
        o_ref[...] = jnp.concatenate([lo, hi], axis=1)

    return pl.pallas_call(
        body,
        grid=(TC_GRID,),
        in_specs=[pl.BlockSpec((D_MODEL, TC_BW), lambda i: (0, i))],
        out_specs=pl.BlockSpec((TC_BW // 2, 128), lambda i: (i, 0)),
        out_shape=jax.ShapeDtypeStruct((T2_ROWS, 128), jnp.float32),
    )(tableT)


GPC = CB // 128
N_G = N_BATCH // 128


@jax.jit
def _embed(idx_flat, table):
    mesh = plsc.VectorSubcoreMesh(core_axis_name="c", subcore_axis_name="s")

    @functools.partial(
        pl.kernel,
        mesh=mesh,
        out_type=jax.ShapeDtypeStruct((N_COLS, 8, N_G, 1024), jnp.float32),
        scratch_types=[
            pltpu.VMEM((CB,), jnp.int32),
            pltpu.VMEM((CB,), jnp.int32),
            pltpu.VMEM((CB, D_MODEL), jnp.float32),
            pltpu.VMEM((CB, D_MODEL), jnp.float32),
            pltpu.VMEM((8 * GPC, 1024), jnp.float32),
            pltpu.VMEM((8 * GPC, 1024), jnp.float32),
            *[pltpu.SemaphoreType.DMA for _ in range(6)],
        ],
        compiler_params=pltpu.CompilerParams(
            use_tc_tiling_on_sc=False, needs_layout_passes=False
        ),
    )
    def emb(idx_hbm, table_hbm, out_hbm,
            idx0, idx1, rows0, rows1, tr0, tr1,
            is0, is1, gs0, gs1, os0, os1):
        idx = (idx0, idx1)
        rows = (rows0, rows1)
        tr = (tr0, tr1)
        isem = (is0, is1)
        gsem = (gs0, gs1)
        osem = (os0, os1)

        wid = lax.axis_index("s") * NUM_CORES + lax.axis_index("c")
        ibase = wid * PER_TILE * CB

        iota = lax.iota(jnp.int32, 16)
        colbase = (iota & 7) << 7
        iotahi = iota >> 3

        def start_idx(b, i):
            pltpu.async_copy(
                idx_hbm.at[pl.ds(ibase + i * CB, CB)], idx[b], isem[b]
            )

        def wait_idx(b):
            pltpu.make_async_copy(
                idx_hbm.at[pl.ds(ibase, CB)], idx[b], isem[b]
            ).wait()

        def start_gather(b, i):
            del i
            pltpu.async_copy(table_hbm.at[idx[b]], rows[b], gsem[b])

        def wait_gather(b):
            pltpu.make_async_copy(
                out_hbm.at[0, 0, pl.ds(0, 8 * GPC), :], rows[b], gsem[b]
            ).wait()

        def transpose_scale(b):
            rb = rows[b]
            tb = tr[b]

            gb_bits = (CB // 16).bit_length() - 1

            @plsc.parallel_loop(0, (D_MODEL // 16) * CB, unroll=4)
            def _(it):
                s = it & 15
                gb = (it >> 4) & (CB // 16 - 1)
                jb = it >> (4 + gb_bits)
                skew = (iota + s) & 15
                rr = gb * 16 + skew
                cc = jb * 16 + iota
                v = plsc.load_gather(rb, [rr, cc])
                row = (gb >> 3) * 8 + jb * 2 + iotahi
                col = colbase + ((gb & 7) * 16 + skew)
                plsc.store_scatter(tb, [row, col], v)

        def start_store(b, i):
            ig = wid * PER_TILE + i
            c = ig // CHUNKS_PER_COL
            k = ig % CHUNKS_PER_COL
            for g in range(GPC):
                pltpu.async_copy(
                    tr[b].at[pl.ds(g * 8, 8), :],
                    out_hbm.at[c, :, GPC * k + g, :],
                    osem[b],
                )

        def wait_store(b):
            for _ in range(GPC):
                pltpu.make_async_copy(
                    tr[b].at[pl.ds(0, 8), :], out_hbm.at[0, :, 0, :], osem[b]
                ).wait()

        pltpu.sync_copy(idx_hbm.at[pl.ds(ibase, CB)], idx0)
        start_idx(1, 1)
        start_gather(0, 0)

        def chunk_body(i, b, first, do_next_gather, do_idx_load):
            wait_gather(b)
            if do_next_gather:
                wait_idx(1 - b)
                start_gather(1 - b, i + 1)
            if do_idx_load:
                start_idx(b, i + 2)
            if not first:
                wait_store(b)
            transpose_scale(b)
            start_store(b, i)

        chunk_body(0, 0, True, True, True)
        chunk_body(1, 1, True, True, True)

        def outer(p, carry):
            i0 = p * 2
            chunk_body(i0, 0, False, True, True)
            chunk_body(i0 + 1, 1, False, True, True)
            return carry

        lax.fori_loop(1, PER_TILE // 2 - 1, outer, 0)

        chunk_body(PER_TILE - 2, 0, False, True, False)
        chunk_body(PER_TILE - 1, 1, False, False, False)

        wait_store(0)
        wait_store(1)

    return emb(idx_flat, table)


@jax.jit
def _full(x, table):
    xT = x.T
    v = xT.reshape(N_COLS * N_BATCH).astype(jnp.int32)
    half_sh = (TC_BW // 2).bit_length() - 1
    idx_flat = (
        (v & -TC_BW) + ((v & (TC_BW // 2 - 1)) << 1) + ((v >> half_sh) & 1)
    )
    table_rm = _tc_transpose_scale(table.T).reshape(2 * T2_ROWS, D_MODEL)
    outP = _embed(idx_flat, table_rm)
    y = (
        outP.reshape(N_COLS, 8, N_G, 8, 128)
        .transpose(0, 1, 3, 2, 4)
        .reshape(N_COLS, D_MODEL, N_BATCH)
    )
    return y.transpose(2, 0, 1)


def kernel(x, table):
    return _full(x, table)
